# Initial kernel scaffold; baseline (speedup 1.0000x reference)
import jax
import jax.numpy as jnp
from jax.experimental import pallas as pl


def kernel(input_energies, z, mean, stddev):
    raise NotImplementedError("write your pallas kernel here")


# speedup vs baseline: 136.1740x; 136.1740x over previous
import functools

import jax
import jax.numpy as jnp
from jax import lax
from jax.experimental import pallas as pl
from jax.experimental.pallas import tpu as pltpu, tpu_sc as plsc

_LANES = 16
_TABLE_PAD = 128


@functools.cache
def _make_sc_kernel(n_pad: int, num_workers: int, b_per_w: int):
    mesh = plsc.VectorSubcoreMesh(core_axis_name="c", subcore_axis_name="s")
    num_cores = plsc.get_sparse_core_info().num_cores

    @functools.partial(
        pl.kernel,
        out_type=jax.ShapeDtypeStruct((n_pad,), jnp.float32),
        mesh=mesh,
        scratch_types=[
            pltpu.VMEM((b_per_w,), jnp.int32),
            pltpu.VMEM((b_per_w,), jnp.float32),
            pltpu.VMEM((b_per_w,), jnp.float32),
            pltpu.VMEM((_TABLE_PAD,), jnp.float32),
            pltpu.VMEM((_TABLE_PAD,), jnp.float32),
        ],
        compiler_params=pltpu.CompilerParams(needs_layout_passes=False),
    )
    def body(e_hbm, z_hbm, mean_hbm, std_hbm, out_hbm,
             z_v, e_v, y_v, mean_v, std_v):
        wid = lax.axis_index("s") * num_cores + lax.axis_index("c")
        base = wid * b_per_w
        pltpu.sync_copy(mean_hbm, mean_v)
        pltpu.sync_copy(std_hbm, std_v)
        pltpu.sync_copy(z_hbm.at[pl.ds(base, b_per_w)], z_v)
        pltpu.sync_copy(e_hbm.at[pl.ds(base, b_per_w)], e_v)

        def step(i, carry):
            off = i * _LANES
            zz = z_v[pl.ds(off, _LANES)]
            s = plsc.load_gather(std_v, [zz])
            m = plsc.load_gather(mean_v, [zz])
            y_v[pl.ds(off, _LANES)] = e_v[pl.ds(off, _LANES)] * s + m
            return carry

        lax.fori_loop(0, b_per_w // _LANES, step, 0)
        pltpu.sync_copy(y_v, out_hbm.at[pl.ds(base, b_per_w)])

    return body


def kernel(input_energies, z, mean, stddev):
    n = input_energies.shape[0]
    num_workers = 32
    chunk = _LANES * num_workers
    n_pad = -(-n // chunk) * chunk
    b_per_w = n_pad // num_workers

    e = input_energies.reshape(-1)
    zi = z.astype(jnp.int32)
    e_p = jnp.pad(e, (0, n_pad - n))
    z_p = jnp.pad(zi, (0, n_pad - n))
    mean_p = jnp.pad(mean, (0, _TABLE_PAD - mean.shape[0]))
    std_p = jnp.pad(stddev, (0, _TABLE_PAD - stddev.shape[0]))

    out = _make_sc_kernel(n_pad, num_workers, b_per_w)(e_p, z_p, mean_p, std_p)
    return out[:n].reshape(n, 1)


# speedup vs baseline: 141.4356x; 1.0386x over previous
import functools

import jax
import jax.numpy as jnp
from jax import lax
from jax.experimental import pallas as pl
from jax.experimental.pallas import tpu as pltpu, tpu_sc as plsc

_LANES = 16
_UNROLL = 8
_TABLE_PAD = 128


@functools.cache
def _make_sc_kernel(n_pad: int, num_workers: int, b_per_w: int):
    mesh = plsc.VectorSubcoreMesh(core_axis_name="c", subcore_axis_name="s")
    num_cores = plsc.get_sparse_core_info().num_cores

    @functools.partial(
        pl.kernel,
        out_type=jax.ShapeDtypeStruct((n_pad,), jnp.float32),
        mesh=mesh,
        scratch_types=[
            pltpu.VMEM((b_per_w,), jnp.int32),
            pltpu.VMEM((b_per_w,), jnp.float32),
            pltpu.VMEM((b_per_w,), jnp.float32),
            pltpu.VMEM((_TABLE_PAD,), jnp.float32),
            pltpu.VMEM((_TABLE_PAD,), jnp.float32),
        ],
        compiler_params=pltpu.CompilerParams(needs_layout_passes=False),
    )
    def body(e_hbm, z_hbm, mean_hbm, std_hbm, out_hbm,
             z_v, e_v, y_v, mean_v, std_v):
        wid = lax.axis_index("s") * num_cores + lax.axis_index("c")
        base = wid * b_per_w
        pltpu.sync_copy(mean_hbm, mean_v)
        pltpu.sync_copy(std_hbm, std_v)
        pltpu.sync_copy(z_hbm.at[pl.ds(base, b_per_w)], z_v)
        pltpu.sync_copy(e_hbm.at[pl.ds(base, b_per_w)], e_v)

        @plsc.parallel_loop(0, b_per_w, step=_LANES, unroll=_UNROLL)
        def _(off):
            zz = z_v[pl.ds(off, _LANES)]
            s = plsc.load_gather(std_v, [zz])
            m = plsc.load_gather(mean_v, [zz])
            y_v[pl.ds(off, _LANES)] = e_v[pl.ds(off, _LANES)] * s + m
        pltpu.sync_copy(y_v, out_hbm.at[pl.ds(base, b_per_w)])

    return body


def kernel(input_energies, z, mean, stddev):
    n = input_energies.shape[0]
    num_workers = 32
    chunk = _LANES * _UNROLL * num_workers
    n_pad = -(-n // chunk) * chunk
    b_per_w = n_pad // num_workers

    e = input_energies.reshape(-1)
    zi = z.astype(jnp.int32)
    e_p = jnp.pad(e, (0, n_pad - n))
    z_p = jnp.pad(zi, (0, n_pad - n))
    mean_p = jnp.pad(mean, (0, _TABLE_PAD - mean.shape[0]))
    std_p = jnp.pad(stddev, (0, _TABLE_PAD - stddev.shape[0]))

    out = _make_sc_kernel(n_pad, num_workers, b_per_w)(e_p, z_p, mean_p, std_p)
    return out[:n].reshape(n, 1)


# speedup vs baseline: 158.8114x; 1.1229x over previous
import functools

import jax
import jax.numpy as jnp
from jax import lax
from jax.experimental import pallas as pl
from jax.experimental.pallas import tpu as pltpu, tpu_sc as plsc

_LANES = 16
_UNROLL = 8


@functools.cache
def _make_sc_kernel(n: int, table_n: int):
    info = plsc.get_sparse_core_info()
    num_cores = info.num_cores
    num_workers = num_cores * info.num_subcores
    step = _LANES * _UNROLL
    b_per_w = -(-n // (num_workers * step)) * step
    assert n % 8 == 0 and b_per_w <= n
    mesh = plsc.VectorSubcoreMesh(core_axis_name="c", subcore_axis_name="s")

    @functools.partial(
        pl.kernel,
        out_type=jax.ShapeDtypeStruct((n,), jnp.float32),
        mesh=mesh,
        scratch_types=[
            pltpu.VMEM((b_per_w,), jnp.int32),
            pltpu.VMEM((b_per_w,), jnp.float32),
            pltpu.VMEM((b_per_w,), jnp.float32),
            pltpu.VMEM((table_n,), jnp.float32),
            pltpu.VMEM((table_n,), jnp.float32),
        ],
        compiler_params=pltpu.CompilerParams(needs_layout_passes=False),
    )
    def body(e_hbm, z_hbm, mean_hbm, std_hbm, out_hbm,
             z_v, e_v, y_v, mean_v, std_v):
        wid = lax.axis_index("s") * num_cores + lax.axis_index("c")
        base = jnp.minimum(wid * b_per_w, n - b_per_w)
        pltpu.sync_copy(mean_hbm, mean_v)
        pltpu.sync_copy(std_hbm, std_v)
        pltpu.sync_copy(z_hbm.at[pl.ds(base, b_per_w)], z_v)
        pltpu.sync_copy(e_hbm.at[pl.ds(base, b_per_w)], e_v)

        @plsc.parallel_loop(0, b_per_w, step=_LANES, unroll=_UNROLL)
        def _(off):
            zz = z_v[pl.ds(off, _LANES)]
            s = plsc.load_gather(std_v, [zz])
            m = plsc.load_gather(mean_v, [zz])
            y_v[pl.ds(off, _LANES)] = e_v[pl.ds(off, _LANES)] * s + m

        pltpu.sync_copy(y_v, out_hbm.at[pl.ds(base, b_per_w)])

    return body


def kernel(input_energies, z, mean, stddev):
    n = input_energies.shape[0]
    e = input_energies.reshape(n)
    zi = z.astype(jnp.int32)
    out = _make_sc_kernel(n, mean.shape[0])(e, zi, mean, stddev)
    return out.reshape(n, 1)


# speedup vs baseline: 249.4986x; 1.5710x over previous
import functools

import jax
import jax.numpy as jnp
from jax import lax
from jax.experimental import pallas as pl
from jax.experimental.pallas import tpu as pltpu, tpu_sc as plsc

_LANES = 16
_UNROLL = 8


@functools.cache
def _make_sc_kernel(n: int, table_n: int):
    info = plsc.get_sparse_core_info()
    num_cores = info.num_cores
    num_workers = num_cores * info.num_subcores
    step = _LANES * _UNROLL
    b_per_w = -(-n // (num_workers * step)) * step
    assert n % 8 == 0 and b_per_w <= n
    mesh = plsc.VectorSubcoreMesh(core_axis_name="c", subcore_axis_name="s")

    @functools.partial(
        pl.kernel,
        out_type=jax.ShapeDtypeStruct((n,), jnp.float32),
        mesh=mesh,
        scratch_types=[
            pltpu.VMEM((b_per_w,), jnp.int32),
            pltpu.VMEM((b_per_w,), jnp.float32),
            pltpu.VMEM((b_per_w,), jnp.float32),
            pltpu.VMEM((table_n,), jnp.float32),
            pltpu.VMEM((table_n,), jnp.float32),
        ],
        compiler_params=pltpu.CompilerParams(needs_layout_passes=False),
    )
    def body(e_hbm, z_hbm, mean_hbm, std_hbm, out_hbm,
             z_v, e_v, y_v, mean_v, std_v):
        wid = lax.axis_index("s") * num_cores + lax.axis_index("c")
        base = jnp.minimum(wid * b_per_w, n - b_per_w)
        pltpu.sync_copy(mean_hbm, mean_v)
        pltpu.sync_copy(std_hbm, std_v)
        pltpu.sync_copy(z_hbm.at[pl.ds(base, b_per_w)], z_v)
        pltpu.sync_copy(e_hbm.at[pl.ds(base, b_per_w)], e_v)

        @plsc.parallel_loop(0, b_per_w, step=_LANES, unroll=_UNROLL)
        def _(off):
            zz = z_v[pl.ds(off, _LANES)]
            s = plsc.load_gather(std_v, [zz])
            m = plsc.load_gather(mean_v, [zz])
            y_v[pl.ds(off, _LANES)] = e_v[pl.ds(off, _LANES)] * s + m

        pltpu.sync_copy(y_v, out_hbm.at[pl.ds(base, b_per_w)])

    return body


def kernel(input_energies, z, mean, stddev):
    n = input_energies.shape[0]
    n2 = -(-n // 1024) * 1024
    e2 = jnp.pad(input_energies, ((0, n2 - n), (0, 0)))
    z2 = jnp.pad(z.astype(jnp.int32), (0, n2 - n))
    e_flat = e2.reshape(n2)
    out = _make_sc_kernel(n2, mean.shape[0])(e_flat, z2, mean, stddev)
    return out.reshape(n2, 1)[:n]
